# baseline (speedup 1.0000x reference)
import functools

import jax
import jax.numpy as jnp
from jax.experimental import pallas as pl
from jax.experimental.pallas import tpu as pltpu


def _pe_add_kernel(x_ref, table_ref, out_ref, pe_ref, *, seq_len, vocab, max_rel):
    b = pl.program_id(0)
    s = pl.program_id(1)

    @pl.when(jnp.logical_and(b == 0, s == 0))
    def _compute_pe():
        S, V, R = seq_len, vocab, max_rel
        i = jax.lax.broadcasted_iota(jnp.int32, (S, V), 0)
        k = jax.lax.broadcasted_iota(jnp.int32, (S, V), 1)
        rel = k - R
        counts = jnp.logical_and(rel >= -i, rel <= S - 1 - i).astype(jnp.float32)
        n_lo = jnp.maximum(i - (R - 1), 0).astype(jnp.float32)
        n_hi = jnp.maximum(S - i - R, 0).astype(jnp.float32)
        counts = jnp.where(k == 0, n_lo, counts)
        counts = jnp.where(k == V - 1, n_hi, counts)
        pe_ref[...] = jnp.dot(
            counts,
            table_ref[...],
            preferred_element_type=jnp.float32,
        ) * (1.0 / S)

    blk = out_ref.shape[1]
    out_ref[...] = x_ref[...] + pe_ref[pl.ds(s * blk, blk), :][None]


def kernel(x, table):
    B, S, D = x.shape
    V, _ = table.shape
    R = (V - 1) // 2
    SBLK = 256
    body = functools.partial(_pe_add_kernel, seq_len=S, vocab=V, max_rel=R)
    return pl.pallas_call(
        body,
        grid=(B, S // SBLK),
        in_specs=[
            pl.BlockSpec((1, SBLK, D), lambda b, s: (b, s, 0)),
            pl.BlockSpec((V, D), lambda b, s: (0, 0)),
        ],
        out_specs=pl.BlockSpec((1, SBLK, D), lambda b, s: (b, s, 0)),
        out_shape=jax.ShapeDtypeStruct((B, S, D), x.dtype),
        scratch_shapes=[pltpu.VMEM((S, D), jnp.float32)],
    )(x, table)


# speedup vs baseline: 1.6654x; 1.6654x over previous
import functools

import jax
import jax.numpy as jnp
from jax.experimental import pallas as pl
from jax.experimental.pallas import tpu as pltpu


def _pe_add_kernel(x_ref, table_ref, out_ref, pe_ref, *, seq_len, vocab, max_rel):
    b = pl.program_id(0)

    @pl.when(b == 0)
    def _compute_pe():
        S, V, R = seq_len, vocab, max_rel
        i = jax.lax.broadcasted_iota(jnp.int32, (S, V), 0)
        k = jax.lax.broadcasted_iota(jnp.int32, (S, V), 1)
        rel = k - R
        counts = jnp.logical_and(rel >= -i, rel <= S - 1 - i).astype(jnp.float32)
        n_lo = jnp.maximum(i - (R - 1), 0).astype(jnp.float32)
        n_hi = jnp.maximum(S - i - R, 0).astype(jnp.float32)
        counts = jnp.where(k == 0, n_lo, counts)
        counts = jnp.where(k == V - 1, n_hi, counts)
        pe_ref[...] = jnp.dot(
            counts,
            table_ref[...],
            preferred_element_type=jnp.float32,
        ) * (1.0 / S)

    out_ref[...] = x_ref[...] + pe_ref[...][None]


def kernel(x, table):
    B, S, D = x.shape
    V, _ = table.shape
    R = (V - 1) // 2
    body = functools.partial(_pe_add_kernel, seq_len=S, vocab=V, max_rel=R)
    return pl.pallas_call(
        body,
        grid=(B,),
        in_specs=[
            pl.BlockSpec((1, S, D), lambda b: (b, 0, 0)),
            pl.BlockSpec((V, D), lambda b: (0, 0)),
        ],
        out_specs=pl.BlockSpec((1, S, D), lambda b: (b, 0, 0)),
        out_shape=jax.ShapeDtypeStruct((B, S, D), x.dtype),
        scratch_shapes=[pltpu.VMEM((S, D), jnp.float32)],
    )(x, table)


# speedup vs baseline: 1.6661x; 1.0004x over previous
import functools

import jax
import jax.numpy as jnp
from jax.experimental import pallas as pl
from jax.experimental.pallas import tpu as pltpu


def _pe_add_kernel(x_ref, table_ref, out_ref, pe_ref, *, seq_len, vocab, max_rel):
    b = pl.program_id(0)

    @pl.when(b == 0)
    def _compute_pe():
        S, V, R = seq_len, vocab, max_rel
        i = jax.lax.broadcasted_iota(jnp.int32, (S, V), 0)
        k = jax.lax.broadcasted_iota(jnp.int32, (S, V), 1)
        rel = k - R
        counts = jnp.logical_and(rel >= -i, rel <= S - 1 - i).astype(jnp.float32)
        n_lo = jnp.maximum(i - (R - 1), 0).astype(jnp.float32)
        n_hi = jnp.maximum(S - i - R, 0).astype(jnp.float32)
        counts = jnp.where(k == 0, n_lo, counts)
        counts = jnp.where(k == V - 1, n_hi, counts)
        pe_ref[...] = jnp.dot(
            counts.astype(jnp.bfloat16),
            table_ref[...].astype(jnp.bfloat16),
            preferred_element_type=jnp.float32,
        ) * (1.0 / S)

    out_ref[...] = x_ref[...] + pe_ref[...][None]


def kernel(x, table):
    B, S, D = x.shape
    V, _ = table.shape
    R = (V - 1) // 2
    body = functools.partial(_pe_add_kernel, seq_len=S, vocab=V, max_rel=R)
    return pl.pallas_call(
        body,
        grid=(B,),
        in_specs=[
            pl.BlockSpec((1, S, D), lambda b: (b, 0, 0)),
            pl.BlockSpec((V, D), lambda b: (0, 0)),
        ],
        out_specs=pl.BlockSpec((1, S, D), lambda b: (b, 0, 0)),
        out_shape=jax.ShapeDtypeStruct((B, S, D), x.dtype),
        scratch_shapes=[pltpu.VMEM((S, D), jnp.float32)],
    )(x, table)
